# baseline (speedup 1.0000x reference)
import jax
import jax.numpy as jnp
from jax.experimental import pallas as pl
from jax.experimental.pallas import tpu as pltpu

B, C, H, W = 256, 3, 224, 224
OW, OH = 32, 32
WIN = 40


def _body(ids_ref, cd_ref, obj_ref, bg_ref, tab_ref, out_ref):
    x = cd_ref[0]
    y = cd_ref[1]
    out_ref[...] = bg_ref[...]
    x8 = pl.multiple_of((x // 8) * 8, 8)
    dx = x - x8
    m = tab_ref[0]
    mp = jnp.pad(m, ((0, WIN - OW), (0, W - OH)))
    mp = pltpu.roll(mp, dx, axis=0)
    mp = pltpu.roll(mp, y, axis=1)
    op = jnp.pad(obj_ref[0], ((0, 0), (0, WIN - OW), (0, W - OH)))
    op = pltpu.roll(op, dx, axis=1)
    op = pltpu.roll(op, y, axis=2)
    win = bg_ref[0, :, pl.ds(x8, WIN), :]
    out_ref[0, :, pl.ds(x8, WIN), :] = (1.0 - mp)[None] * win + mp[None] * op


def kernel(obj, bg, coord, obj_id, table):
    table3 = table.reshape(table.shape[0], OW, OH)
    grid_spec = pltpu.PrefetchScalarGridSpec(
        num_scalar_prefetch=2,
        grid=(B,),
        in_specs=[
            pl.BlockSpec((1, C, OW, OH), lambda b, ids, cd: (b, 0, 0, 0)),
            pl.BlockSpec((1, C, H, W), lambda b, ids, cd: (b, 0, 0, 0)),
            pl.BlockSpec((1, OW, OH), lambda b, ids, cd: (ids[b], 0, 0)),
        ],
        out_specs=pl.BlockSpec((1, C, H, W), lambda b, ids, cd: (b, 0, 0, 0)),
    )
    return pl.pallas_call(
        _body,
        grid_spec=grid_spec,
        out_shape=jax.ShapeDtypeStruct((B, C, H, W), jnp.float32),
    )(obj_id, coord, obj, bg, table3)
